# Initial kernel scaffold; baseline (speedup 1.0000x reference)
import jax
import jax.numpy as jnp
from jax.experimental import pallas as pl


def kernel(x, edge_index, comm_ids, comm_emb, W1, b1, W2, b2):
    raise NotImplementedError("write your pallas kernel here")


# speedup vs baseline: 11.4435x; 11.4435x over previous
import functools

import jax
import jax.numpy as jnp
from jax import lax
from jax.experimental import pallas as pl
from jax.experimental.pallas import tpu as pltpu
from jax.experimental.pallas import tpu_sc as plsc

N_NODES = 10000
E = 320000
IN_CH = 128
HID = 128
OUT = 64
COMM_DIM = 8
CEMB_PAD = 128

NC, NS = 2, 16
NW = NC * NS
EPT = E // NW
CHUNK = 80
NCHUNK = EPT // CHUNK
NP = 10240
ROWS_PT = NP // NS
C_PAD = NP
CROWS = C_PAD // NW
NCCHUNK = CROWS // CHUNK

_mesh = plsc.VectorSubcoreMesh(core_axis_name="c", subcore_axis_name="s")


def _prep_body(dst_hbm, cids_hbm, cemb_hbm, ones_hbm, z16_hbm,
               deg_out, c_out,
               ones_v, idx_v, rows_v, acc, sem):
    c = lax.axis_index("c")
    s = lax.axis_index("s")
    w = c * NS + s
    pltpu.sync_copy(z16_hbm, acc.at[pl.ds(s * ROWS_PT, ROWS_PT), :])
    pltpu.sync_copy(ones_hbm, ones_v)
    plsc.subcore_barrier()

    ebase = c * (E // NC) + s * EPT

    def deg_step(i, _):
        pltpu.sync_copy(dst_hbm.at[pl.ds(ebase + i * CHUNK, CHUNK)], idx_v)
        pltpu.sync_copy(ones_v, acc.at[idx_v], add=True)
        return 0

    lax.fori_loop(0, NCHUNK, deg_step, 0)

    cbase = w * CROWS

    def comm_step(i, _):
        b = cbase + i * CHUNK
        pltpu.sync_copy(cids_hbm.at[pl.ds(b, CHUNK)], idx_v)
        pltpu.async_copy(cemb_hbm.at[idx_v], rows_v, sem).wait()
        pltpu.sync_copy(rows_v, c_out.at[pl.ds(b, CHUNK), :])
        return 0

    lax.fori_loop(0, NCCHUNK, comm_step, 0)

    plsc.subcore_barrier()
    pltpu.sync_copy(acc.at[pl.ds(s * ROWS_PT, ROWS_PT), :],
                    deg_out.at[c, pl.ds(s * ROWS_PT, ROWS_PT), :])


_prep = pl.kernel(
    _prep_body,
    out_type=(jax.ShapeDtypeStruct((NC, NP, 16), jnp.float32),
              jax.ShapeDtypeStruct((C_PAD, CEMB_PAD), jnp.float32)),
    mesh=_mesh,
    scratch_types=[
        pltpu.VMEM((CHUNK, 16), jnp.float32),
        pltpu.VMEM((CHUNK,), jnp.int32),
        pltpu.VMEM((CHUNK, CEMB_PAD), jnp.float32),
        pltpu.VMEM_SHARED((NP, 16), jnp.float32),
        pltpu.SemaphoreType.DMA,
    ],
)


def _edge_body(p_hbm, src_hbm, dst_hbm, zrow_hbm,
               agg_out, src_v, dst_v, rows_v, acc, sem):
    c = lax.axis_index("c")
    s = lax.axis_index("s")
    pltpu.sync_copy(zrow_hbm, acc.at[pl.ds(s * ROWS_PT, ROWS_PT), :])
    plsc.subcore_barrier()

    ebase = c * (E // NC) + s * EPT

    def step(i, _):
        b = ebase + i * CHUNK
        pltpu.sync_copy(src_hbm.at[pl.ds(b, CHUNK)], src_v)
        pltpu.sync_copy(dst_hbm.at[pl.ds(b, CHUNK)], dst_v)
        pltpu.async_copy(p_hbm.at[src_v], rows_v, sem).wait()
        pltpu.sync_copy(rows_v, acc.at[dst_v], add=True)
        return 0

    lax.fori_loop(0, NCHUNK, step, 0)

    plsc.subcore_barrier()
    pltpu.sync_copy(acc.at[pl.ds(s * ROWS_PT, ROWS_PT), :],
                    agg_out.at[c, pl.ds(s * ROWS_PT, ROWS_PT), :])


def _make_edge_kernel(width):
    return pl.kernel(
        _edge_body,
        out_type=jax.ShapeDtypeStruct((NC, NP, width), jnp.float32),
        mesh=_mesh,
        scratch_types=[
            pltpu.VMEM((CHUNK,), jnp.int32),
            pltpu.VMEM((CHUNK,), jnp.int32),
            pltpu.VMEM((CHUNK, width), jnp.float32),
            pltpu.VMEM_SHARED((NP, width), jnp.float32),
            pltpu.SemaphoreType.DMA,
        ],
    )


_edge = _make_edge_kernel(HID)


def _tc1_body(x_ref, c_ref, w1_ref, deg_ref, p1_ref):
    dinv = lax.rsqrt(deg_ref[0, 0:N_NODES, 0:1] + deg_ref[1, 0:N_NODES, 0:1])
    h = jnp.dot(x_ref[...], w1_ref[0:IN_CH, :],
                preferred_element_type=jnp.float32)
    h = h + jnp.dot(c_ref[0:N_NODES, 0:COMM_DIM],
                    w1_ref[IN_CH:IN_CH + COMM_DIM, :],
                    preferred_element_type=jnp.float32)
    p1_ref[0:N_NODES, :] = h * dinv


_tc1 = pl.pallas_call(
    _tc1_body,
    out_shape=jax.ShapeDtypeStruct((NP, HID), jnp.float32),
)


def _tc2_body(agg_ref, p1_ref, deg_ref, b1_ref, q_ref):
    dinv = lax.rsqrt(deg_ref[0, 0:N_NODES, 0:1] + deg_ref[1, 0:N_NODES, 0:1])
    t = ((agg_ref[0, 0:N_NODES, :] + agg_ref[1, 0:N_NODES, :]
          + p1_ref[0:N_NODES, :]) * dinv + b1_ref[...])
    q_ref[0:N_NODES, :] = jnp.maximum(t, 0.0) * dinv


_tc2 = pl.pallas_call(
    _tc2_body,
    out_shape=jax.ShapeDtypeStruct((NP, HID), jnp.float32),
)


def _tc3_body(agg_ref, q_ref, deg_ref, w2_ref, b2_ref, out_ref):
    dinv = lax.rsqrt(deg_ref[0, 0:N_NODES, 0:1] + deg_ref[1, 0:N_NODES, 0:1])
    s = (agg_ref[0, 0:N_NODES, :] + agg_ref[1, 0:N_NODES, :]
         + q_ref[0:N_NODES, :])
    out_ref[...] = (jnp.dot(s, w2_ref[...], preferred_element_type=jnp.float32)
                    * dinv + b2_ref[...])


_tc3 = pl.pallas_call(
    _tc3_body,
    out_shape=jax.ShapeDtypeStruct((N_NODES, OUT), jnp.float32),
)


def kernel(x, edge_index, comm_ids, comm_emb, W1, b1, W2, b2):
    src = edge_index[0].astype(jnp.int32)
    dst = edge_index[1].astype(jnp.int32)
    cids = jnp.pad(comm_ids.astype(jnp.int32), (0, C_PAD - N_NODES))
    cemb = jnp.pad(comm_emb, ((0, 0), (0, CEMB_PAD - COMM_DIM)))
    ones16 = jnp.ones((CHUNK, 16), jnp.float32)
    z16 = jnp.zeros((ROWS_PT, 16), jnp.float32)
    z128 = jnp.zeros((ROWS_PT, HID), jnp.float32)

    deg2, cpad = _prep(dst, cids, cemb, ones16, z16)
    p1 = _tc1(x, cpad, W1, deg2)
    agg1 = _edge(p1, src, dst, z128)
    q = _tc2(agg1, p1, deg2, b1.reshape(1, HID))
    aggq = _edge(q, src, dst, z128)
    return _tc3(aggq, q, deg2, W2, b2.reshape(1, OUT))
